# Initial kernel scaffold; baseline (speedup 1.0000x reference)
import jax
import jax.numpy as jnp
from jax.experimental import pallas as pl


def kernel(x, router_w, gate_w, up_w, down_w):
    raise NotImplementedError("write your pallas kernel here")


# speedup vs baseline: 1.7525x; 1.7525x over previous
import functools

import jax
import jax.numpy as jnp
from jax.experimental import pallas as pl
from jax.experimental.pallas import tpu as pltpu

DIM = 1024
NUM_EXPERTS = 8
TOP_K = 2
HIDDEN = 2730
SEQ = 2048
EPAD = 128
H_TILE = 256
H_PAD = 2816
NH = H_PAD // H_TILE


def _router_kernel(x_ref, rw_ref, fullw_ref, lb_ref, z_ref):
    x = x_ref[...]
    rw = rw_ref[...]
    logits = jax.lax.dot_general(
        x, rw, (((1,), (1,)), ((), ())), preferred_element_type=jnp.float32
    )
    lane = jax.lax.broadcasted_iota(jnp.int32, (SEQ, EPAD), 1)
    valid = lane < NUM_EXPERTS
    neg = jnp.float32(-1e30)
    lm = jnp.where(valid, logits, neg)
    m1 = jnp.max(lm, axis=1, keepdims=True)
    i1 = jnp.min(jnp.where(lm == m1, lane, EPAD), axis=1, keepdims=True)
    lm2 = jnp.where(lane == i1, neg, lm)
    m2 = jnp.max(lm2, axis=1, keepdims=True)
    i2 = jnp.min(jnp.where(lm2 == m2, lane, EPAD), axis=1, keepdims=True)
    e2 = jnp.exp(m2 - m1)
    w1 = 1.0 / (1.0 + e2)
    w2 = 1.0 - w1
    fullw = jnp.where(lane == i1, w1, jnp.where(lane == i2, w2, 0.0))
    fullw_ref[...] = fullw
    p = jnp.where(valid, jnp.exp(lm - m1), 0.0)
    s = jnp.sum(p, axis=1, keepdims=True)
    probs = p / s
    usage = jnp.sum(probs, axis=0, keepdims=True) / SEQ
    lb_ref[...] = NUM_EXPERTS * jnp.sum(usage * usage, keepdims=True)
    zvec = jnp.log(s) + m1
    z_ref[...] = jnp.sum(zvec * zvec, axis=0, keepdims=True) / SEQ


def _expert_kernel(x_ref, fullw_ref, gw_ref, uw_ref, dw_ref, out_ref):
    e = pl.program_id(0)
    h = pl.program_id(1)
    x = x_ref[...]
    gw = gw_ref[0]
    uw = uw_ref[0]
    dw = dw_ref[0]
    g = jax.lax.dot_general(
        x, gw, (((1,), (1,)), ((), ())), preferred_element_type=jnp.float32
    )
    u = jax.lax.dot_general(
        x, uw, (((1,), (1,)), ((), ())), preferred_element_type=jnp.float32
    )
    act = (g / (1.0 + jnp.exp(-g))) * u
    lane = jax.lax.broadcasted_iota(jnp.int32, (SEQ, EPAD), 1)
    w = jnp.sum(jnp.where(lane == e, fullw_ref[...], 0.0), axis=1, keepdims=True)
    act = act * w
    eo = jax.lax.dot_general(
        act, dw, (((1,), (1,)), ((), ())), preferred_element_type=jnp.float32
    )

    @pl.when((e == 0) & (h == 0))
    def _():
        out_ref[...] = eo

    @pl.when((e > 0) | (h > 0))
    def _():
        out_ref[...] = out_ref[...] + eo


def kernel(x, router_w, gate_w, up_w, down_w):
    B, S, D = x.shape
    x_flat = x.reshape(S, D)
    rw_pad = jnp.pad(router_w, ((0, EPAD - NUM_EXPERTS), (0, 0)))
    gw_pad = jnp.pad(gate_w, ((0, 0), (0, H_PAD - HIDDEN), (0, 0)))
    uw_pad = jnp.pad(up_w, ((0, 0), (0, H_PAD - HIDDEN), (0, 0)))
    dw_pad = jnp.pad(down_w, ((0, 0), (0, 0), (0, H_PAD - HIDDEN)))

    fullw, lb, zl = pl.pallas_call(
        _router_kernel,
        out_shape=(
            jax.ShapeDtypeStruct((SEQ, EPAD), jnp.float32),
            jax.ShapeDtypeStruct((1, 1), jnp.float32),
            jax.ShapeDtypeStruct((1, 1), jnp.float32),
        ),
    )(x_flat, rw_pad)

    out = pl.pallas_call(
        _expert_kernel,
        grid=(NUM_EXPERTS, NH),
        in_specs=[
            pl.BlockSpec((S, D), lambda e, h: (0, 0)),
            pl.BlockSpec((S, EPAD), lambda e, h: (0, 0)),
            pl.BlockSpec((1, H_TILE, D), lambda e, h: (e, h, 0)),
            pl.BlockSpec((1, H_TILE, D), lambda e, h: (e, h, 0)),
            pl.BlockSpec((1, D, H_TILE), lambda e, h: (e, 0, h)),
        ],
        out_specs=pl.BlockSpec((S, D), lambda e, h: (0, 0)),
        out_shape=jax.ShapeDtypeStruct((S, D), jnp.float32),
        compiler_params=pltpu.CompilerParams(
            dimension_semantics=("arbitrary", "arbitrary"),
        ),
    )(x_flat, fullw, gw_pad, uw_pad, dw_pad)

    return (out.reshape(B, S, D), lb.reshape(()), zl.reshape(()))
